# Initial kernel scaffold; baseline (speedup 1.0000x reference)
import jax
import jax.numpy as jnp
from jax.experimental import pallas as pl


def kernel(x, edge_index, W1, b1, W2, b2, Wl, bl):
    raise NotImplementedError("write your pallas kernel here")


# speedup vs baseline: 32.9630x; 32.9630x over previous
import functools

import jax
import jax.numpy as jnp
from jax import lax
from jax.experimental import pallas as pl
from jax.experimental.pallas import tpu as pltpu
from jax.experimental.pallas import tpu_sc as plsc

N = 10000
D_IN = 128
DH = 32
DO = 16
NC = 2
NS = 16
NW = NC * NS
EPW = 320000 // NW
CW = 125
CH = EPW // CW
NPAD = 10240
RPT = NPAD // NS
ZB = 128


def _mesh():
    return plsc.VectorSubcoreMesh(
        core_axis_name="c", subcore_axis_name="s", num_cores=NC, num_subcores=NS
    )


_SC_PARAMS = pltpu.CompilerParams(use_tc_tiling_on_sc=False)


def _sc_degree(dst_r):
    @functools.partial(
        pl.kernel,
        out_type=jax.ShapeDtypeStruct((NC, NPAD, 16), jnp.float32),
        mesh=_mesh(),
        compiler_params=_SC_PARAMS,
        scratch_types=[
            pltpu.VMEM((CH, CW), jnp.int32),
            pltpu.VMEM((ZB, 16), jnp.float32),
            pltpu.VMEM((CW, 16), jnp.float32),
            pltpu.VMEM_SHARED((NPAD, 16), jnp.float32),
        ],
    )
    def k(dst_hbm, out_hbm, idx_v, zbuf_v, ones_v, acc_sh):
        c = lax.axis_index("c")
        s = lax.axis_index("s")
        wid = c * NS + s
        pltpu.sync_copy(dst_hbm.at[wid], idx_v)

        def zrow(i, carry):
            zbuf_v[i, :] = jnp.zeros((16,), jnp.float32)
            return carry

        lax.fori_loop(0, ZB, zrow, 0)

        def orow(i, carry):
            ones_v[i, :] = jnp.full((16,), 1.0, jnp.float32)
            return carry

        lax.fori_loop(0, CW, orow, 0)
        base = s * RPT
        for r in range(RPT // ZB):
            pltpu.sync_copy(zbuf_v, acc_sh.at[pl.ds(base + r * ZB, ZB)])
        plsc.subcore_barrier()

        def chunk(j, carry):
            pltpu.sync_copy(ones_v, acc_sh.at[idx_v.at[j]], add=True)
            return carry

        lax.fori_loop(0, CH, chunk, 0)
        plsc.subcore_barrier()
        pltpu.sync_copy(acc_sh.at[pl.ds(base, RPT)], out_hbm.at[c, pl.ds(base, RPT)])

    return k(dst_r)


def _sc_propagate(hp, src_r, dst_r):
    @functools.partial(
        pl.kernel,
        out_type=jax.ShapeDtypeStruct((NC, NPAD, DH), jnp.float32),
        mesh=_mesh(),
        compiler_params=_SC_PARAMS,
        scratch_types=[
            pltpu.VMEM((CH, CW), jnp.int32),
            pltpu.VMEM((CH, CW), jnp.int32),
            pltpu.VMEM((ZB, DH), jnp.float32),
            pltpu.VMEM((CW, DH), jnp.float32),
            pltpu.VMEM_SHARED((NPAD, DH), jnp.float32),
        ],
    )
    def k(hp_hbm, src_hbm, dst_hbm, out_hbm, sidx_v, didx_v, zbuf_v, rows_v, acc_sh):
        c = lax.axis_index("c")
        s = lax.axis_index("s")
        wid = c * NS + s
        pltpu.sync_copy(src_hbm.at[wid], sidx_v)
        pltpu.sync_copy(dst_hbm.at[wid], didx_v)

        def zrow(i, carry):
            zbuf_v[i, pl.ds(0, 16)] = jnp.zeros((16,), jnp.float32)
            zbuf_v[i, pl.ds(16, 16)] = jnp.zeros((16,), jnp.float32)
            return carry

        lax.fori_loop(0, ZB, zrow, 0)
        base = s * RPT
        for r in range(RPT // ZB):
            pltpu.sync_copy(zbuf_v, acc_sh.at[pl.ds(base + r * ZB, ZB)])
        plsc.subcore_barrier()

        def chunk(j, carry):
            pltpu.sync_copy(hp_hbm.at[sidx_v.at[j]], rows_v)
            pltpu.sync_copy(rows_v, acc_sh.at[didx_v.at[j]], add=True)
            return carry

        lax.fori_loop(0, CH, chunk, 0)
        plsc.subcore_barrier()
        pltpu.sync_copy(acc_sh.at[pl.ds(base, RPT)], out_hbm.at[c, pl.ds(base, RPT)])

    return k(hp, src_r, dst_r)


_R = 2000
_G = N // _R


def _dis_block(d_ref):
    deg = d_ref[0, :, 0:1] + d_ref[1, :, 0:1] + 1.0
    return lax.rsqrt(deg)


def _tc_first(x, W1, degp):
    def body(x_ref, w_ref, d_ref, o_ref):
        dis = _dis_block(d_ref)
        h = jnp.dot(x_ref[...], w_ref[...], preferred_element_type=jnp.float32)
        o_ref[...] = dis * h

    return pl.pallas_call(
        body,
        grid=(_G,),
        in_specs=[
            pl.BlockSpec((_R, D_IN), lambda i: (i, 0)),
            pl.BlockSpec((D_IN, DH), lambda i: (0, 0)),
            pl.BlockSpec((2, _R, 16), lambda i: (0, i, 0)),
        ],
        out_specs=pl.BlockSpec((_R, DH), lambda i: (i, 0)),
        out_shape=jax.ShapeDtypeStruct((N, DH), jnp.float32),
    )(x, W1, degp)


def _tc_mid(degp, sp, hp, W2, b1):
    def body(d_ref, sp_ref, hp_ref, w_ref, b_ref, o_ref):
        dis = _dis_block(d_ref)
        seg = sp_ref[0] + sp_ref[1] + hp_ref[...]
        h1 = jnp.maximum(dis * seg + b_ref[...], 0.0)
        o_ref[...] = dis * jnp.dot(h1, w_ref[...], preferred_element_type=jnp.float32)

    return pl.pallas_call(
        body,
        grid=(_G,),
        in_specs=[
            pl.BlockSpec((2, _R, 16), lambda i: (0, i, 0)),
            pl.BlockSpec((2, _R, DH), lambda i: (0, i, 0)),
            pl.BlockSpec((_R, DH), lambda i: (i, 0)),
            pl.BlockSpec((DH, DH), lambda i: (0, 0)),
            pl.BlockSpec((1, DH), lambda i: (0, 0)),
        ],
        out_specs=pl.BlockSpec((_R, DH), lambda i: (i, 0)),
        out_shape=jax.ShapeDtypeStruct((N, DH), jnp.float32),
    )(degp, sp, hp, W2, b1)


def _tc_last(degp, sp, hp, b2, Wl, bl):
    def body(d_ref, sp_ref, hp_ref, b2_ref, wl_ref, bl_ref, o_ref):
        dis = _dis_block(d_ref)
        seg = sp_ref[0] + sp_ref[1] + hp_ref[...]
        h2 = jnp.maximum(dis * seg + b2_ref[...], 0.0)
        o = jnp.dot(h2, wl_ref[...], preferred_element_type=jnp.float32) + bl_ref[...]
        m = jnp.max(o, axis=1, keepdims=True)
        lse = jnp.log(jnp.sum(jnp.exp(o - m), axis=1, keepdims=True)) + m
        o_ref[...] = o - lse

    return pl.pallas_call(
        body,
        grid=(_G,),
        in_specs=[
            pl.BlockSpec((2, _R, 16), lambda i: (0, i, 0)),
            pl.BlockSpec((2, _R, DH), lambda i: (0, i, 0)),
            pl.BlockSpec((_R, DH), lambda i: (i, 0)),
            pl.BlockSpec((1, DH), lambda i: (0, 0)),
            pl.BlockSpec((DH, DO), lambda i: (0, 0)),
            pl.BlockSpec((1, DO), lambda i: (0, 0)),
        ],
        out_specs=pl.BlockSpec((_R, DO), lambda i: (i, 0)),
        out_shape=jax.ShapeDtypeStruct((N, DO), jnp.float32),
    )(degp, sp, hp, b2, Wl, bl)


def kernel(x, edge_index, W1, b1, W2, b2, Wl, bl):
    ei = edge_index.astype(jnp.int32)
    src_r = ei[0].reshape(NW, CH, CW)
    dst_r = ei[1].reshape(NW, CH, CW)

    degp = _sc_degree(dst_r)
    h1p = _tc_first(x, W1, degp)
    s1p = _sc_propagate(h1p, src_r, dst_r)
    h2p = _tc_mid(degp, s1p, h1p, W2, b1.reshape(1, DH))
    s2p = _sc_propagate(h2p, src_r, dst_r)
    return _tc_last(degp, s2p, h2p, b2.reshape(1, DH), Wl, bl.reshape(1, DO))


# speedup vs baseline: 55.1365x; 1.6727x over previous
import functools

import jax
import jax.numpy as jnp
from jax import lax
from jax.experimental import pallas as pl
from jax.experimental.pallas import tpu as pltpu
from jax.experimental.pallas import tpu_sc as plsc

N = 10000
D_IN = 128
DH = 32
DO = 16
NC = 2
NS = 16
NW = NC * NS
EPW = 320000 // NW
CW = 125
CH = EPW // CW
NPAD = 10240
RPT = NPAD // NS
ZB = 128
NB = 8


def _mesh():
    return plsc.VectorSubcoreMesh(
        core_axis_name="c", subcore_axis_name="s", num_cores=NC, num_subcores=NS
    )


_SC_PARAMS = pltpu.CompilerParams(use_tc_tiling_on_sc=False)


def _sc_degree(dst_r):
    @functools.partial(
        pl.kernel,
        out_type=jax.ShapeDtypeStruct((NC, NPAD, 16), jnp.float32),
        mesh=_mesh(),
        compiler_params=_SC_PARAMS,
        scratch_types=[
            pltpu.VMEM((CH, CW), jnp.int32),
            pltpu.VMEM((ZB, 16), jnp.float32),
            pltpu.VMEM((CW, 16), jnp.float32),
            pltpu.VMEM_SHARED((NPAD, 16), jnp.float32),
            pltpu.SemaphoreType.DMA,
        ],
    )
    def k(dst_hbm, out_hbm, idx_v, zbuf_v, ones_v, acc_sh, sem):
        c = lax.axis_index("c")
        s = lax.axis_index("s")
        wid = c * NS + s
        pltpu.sync_copy(dst_hbm.at[wid], idx_v)

        def zrow(i, carry):
            zbuf_v[i, :] = jnp.zeros((16,), jnp.float32)
            return carry

        lax.fori_loop(0, ZB, zrow, 0)

        def orow(i, carry):
            ones_v[i, :] = jnp.full((16,), 1.0, jnp.float32)
            return carry

        lax.fori_loop(0, CW, orow, 0)
        base = s * RPT
        for r in range(RPT // ZB):
            pltpu.sync_copy(zbuf_v, acc_sh.at[pl.ds(base + r * ZB, ZB)])
        plsc.subcore_barrier()

        def chunk(j, carry):
            pltpu.async_copy(ones_v, acc_sh.at[idx_v.at[j]], sem, add=True)
            return carry

        lax.fori_loop(0, CH, chunk, 0)

        def drain(j, carry):
            pltpu.make_async_copy(ones_v, acc_sh.at[idx_v.at[j]], sem).wait()
            return carry

        lax.fori_loop(0, CH, drain, 0)
        plsc.subcore_barrier()
        pltpu.sync_copy(acc_sh.at[pl.ds(base, RPT)], out_hbm.at[c, pl.ds(base, RPT)])

    return k(dst_r)


def _sc_propagate(hp, src_r, dst_r):
    @functools.partial(
        pl.kernel,
        out_type=jax.ShapeDtypeStruct((NC, NPAD, DH), jnp.float32),
        mesh=_mesh(),
        compiler_params=_SC_PARAMS,
        scratch_types=[
            pltpu.VMEM((CH, CW), jnp.int32),
            pltpu.VMEM((CH, CW), jnp.int32),
            pltpu.VMEM((ZB, DH), jnp.float32),
            pltpu.VMEM((NB, CW, DH), jnp.float32),
            pltpu.VMEM_SHARED((NPAD, DH), jnp.float32),
            pltpu.SemaphoreType.DMA((NB,)),
            pltpu.SemaphoreType.DMA((NB,)),
        ],
    )
    def k(hp_hbm, src_hbm, dst_hbm, out_hbm, sidx_v, didx_v, zbuf_v, rows_v,
          acc_sh, gsem, ssem):
        c = lax.axis_index("c")
        s = lax.axis_index("s")
        wid = c * NS + s
        pltpu.sync_copy(src_hbm.at[wid], sidx_v)
        pltpu.sync_copy(dst_hbm.at[wid], didx_v)

        def zrow(i, carry):
            zbuf_v[i, pl.ds(0, 16)] = jnp.zeros((16,), jnp.float32)
            zbuf_v[i, pl.ds(16, 16)] = jnp.zeros((16,), jnp.float32)
            return carry

        lax.fori_loop(0, ZB, zrow, 0)
        base = s * RPT
        for r in range(RPT // ZB):
            pltpu.sync_copy(zbuf_v, acc_sh.at[pl.ds(base + r * ZB, ZB)])
        plsc.subcore_barrier()

        def gather(i, b):
            pltpu.async_copy(hp_hbm.at[sidx_v.at[i]], rows_v.at[b], gsem.at[b])

        def gather_wait(i, b):
            pltpu.make_async_copy(
                hp_hbm.at[sidx_v.at[i]], rows_v.at[b], gsem.at[b]
            ).wait()

        def scatter(i, b):
            pltpu.async_copy(
                rows_v.at[b], acc_sh.at[didx_v.at[i]], ssem.at[b], add=True
            )

        def scatter_wait(i, b):
            pltpu.make_async_copy(
                rows_v.at[b], acc_sh.at[didx_v.at[i]], ssem.at[b]
            ).wait()

        def step(i, p, wait_prev_scatter, prefetch):
            gather_wait(i, p)
            scatter(i, p)
            if prefetch:
                bp = (p + NB - 1) % NB
                if wait_prev_scatter:
                    scatter_wait(i - 1, bp)
                gather(i + NB - 1, bp)

        for b in range(NB - 1):
            gather(b, b)
        step(0, 0, False, True)
        for p in range(1, NB):
            step(p, p, True, True)

        def group(g, carry):
            for p in range(NB):
                step(g * NB + p, p, True, True)
            return carry

        lax.fori_loop(1, CH // NB - 1, group, 0)
        i0 = CH - NB
        step(i0, 0, True, True)
        for p in range(1, NB):
            step(i0 + p, p, False, False)
        for b in range(NB):
            scatter_wait(i0 + b, b)
        plsc.subcore_barrier()
        pltpu.sync_copy(acc_sh.at[pl.ds(base, RPT)], out_hbm.at[c, pl.ds(base, RPT)])

    return k(hp, src_r, dst_r)


_R = 2000
_G = N // _R


def _dis_block(d_ref):
    deg = d_ref[0, :, 0:1] + d_ref[1, :, 0:1] + 1.0
    return lax.rsqrt(deg)


def _tc_first(x, W1, degp):
    def body(x_ref, w_ref, d_ref, o_ref):
        dis = _dis_block(d_ref)
        h = jnp.dot(x_ref[...], w_ref[...], preferred_element_type=jnp.float32)
        o_ref[...] = dis * h

    return pl.pallas_call(
        body,
        grid=(_G,),
        in_specs=[
            pl.BlockSpec((_R, D_IN), lambda i: (i, 0)),
            pl.BlockSpec((D_IN, DH), lambda i: (0, 0)),
            pl.BlockSpec((2, _R, 16), lambda i: (0, i, 0)),
        ],
        out_specs=pl.BlockSpec((_R, DH), lambda i: (i, 0)),
        out_shape=jax.ShapeDtypeStruct((N, DH), jnp.float32),
    )(x, W1, degp)


def _tc_mid(degp, sp, hp, W2, b1):
    def body(d_ref, sp_ref, hp_ref, w_ref, b_ref, o_ref):
        dis = _dis_block(d_ref)
        seg = sp_ref[0] + sp_ref[1] + hp_ref[...]
        h1 = jnp.maximum(dis * seg + b_ref[...], 0.0)
        o_ref[...] = dis * jnp.dot(h1, w_ref[...], preferred_element_type=jnp.float32)

    return pl.pallas_call(
        body,
        grid=(_G,),
        in_specs=[
            pl.BlockSpec((2, _R, 16), lambda i: (0, i, 0)),
            pl.BlockSpec((2, _R, DH), lambda i: (0, i, 0)),
            pl.BlockSpec((_R, DH), lambda i: (i, 0)),
            pl.BlockSpec((DH, DH), lambda i: (0, 0)),
            pl.BlockSpec((1, DH), lambda i: (0, 0)),
        ],
        out_specs=pl.BlockSpec((_R, DH), lambda i: (i, 0)),
        out_shape=jax.ShapeDtypeStruct((N, DH), jnp.float32),
    )(degp, sp, hp, W2, b1)


def _tc_last(degp, sp, hp, b2, Wl, bl):
    def body(d_ref, sp_ref, hp_ref, b2_ref, wl_ref, bl_ref, o_ref):
        dis = _dis_block(d_ref)
        seg = sp_ref[0] + sp_ref[1] + hp_ref[...]
        h2 = jnp.maximum(dis * seg + b2_ref[...], 0.0)
        o = jnp.dot(h2, wl_ref[...], preferred_element_type=jnp.float32) + bl_ref[...]
        m = jnp.max(o, axis=1, keepdims=True)
        lse = jnp.log(jnp.sum(jnp.exp(o - m), axis=1, keepdims=True)) + m
        o_ref[...] = o - lse

    return pl.pallas_call(
        body,
        grid=(_G,),
        in_specs=[
            pl.BlockSpec((2, _R, 16), lambda i: (0, i, 0)),
            pl.BlockSpec((2, _R, DH), lambda i: (0, i, 0)),
            pl.BlockSpec((_R, DH), lambda i: (i, 0)),
            pl.BlockSpec((1, DH), lambda i: (0, 0)),
            pl.BlockSpec((DH, DO), lambda i: (0, 0)),
            pl.BlockSpec((1, DO), lambda i: (0, 0)),
        ],
        out_specs=pl.BlockSpec((_R, DO), lambda i: (i, 0)),
        out_shape=jax.ShapeDtypeStruct((N, DO), jnp.float32),
    )(degp, sp, hp, b2, Wl, bl)


def kernel(x, edge_index, W1, b1, W2, b2, Wl, bl):
    ei = edge_index.astype(jnp.int32)
    src_r = ei[0].reshape(NW, CH, CW)
    dst_r = ei[1].reshape(NW, CH, CW)

    degp = _sc_degree(dst_r)
    h1p = _tc_first(x, W1, degp)
    s1p = _sc_propagate(h1p, src_r, dst_r)
    h2p = _tc_mid(degp, s1p, h1p, W2, b1.reshape(1, DH))
    s2p = _sc_propagate(h2p, src_r, dst_r)
    return _tc_last(degp, s2p, h2p, b2.reshape(1, DH), Wl, bl.reshape(1, DO))
